# baseline (speedup 1.0000x reference)
import functools

import jax
import jax.numpy as jnp
import numpy as np
from jax.experimental import pallas as pl
from jax.experimental.pallas import tpu as pltpu
from jax.sharding import Mesh, PartitionSpec as P


def _cru_body(uq, H, W, kk, nblk, x_ref, wsq_ref, wg_ref, bias_ref, mask_ref,
              o_ref):
    S = H * W
    pad = kk // 2
    wsq = wsq_ref[...]
    wg = wg_ref[...]
    bias = bias_ref[...]

    for i in range(nblk):
        x = x_ref[i].astype(jnp.bfloat16)

        ul = jnp.dot(wsq, x, preferred_element_type=jnp.float32)
        u = ul[:uq, :].astype(jnp.bfloat16)
        y2 = ul[uq:, :]

        taps = []
        t = 0
        for ky in range(kk):
            for kx in range(kk):
                dy, dx = ky - pad, kx - pad
                if dy == 0 and dx == 0:
                    taps.append(u)
                else:
                    shift = (-(dy * W + dx)) % S
                    rolled = pltpu.roll(u, shift=shift, axis=1)
                    taps.append(rolled * mask_ref[t:t + 1, :])
                t += 1
        ucat = jnp.concatenate(taps, axis=0)

        y1 = jnp.dot(wg, ucat, preferred_element_type=jnp.float32) + bias

        m1 = jnp.mean(y1, axis=1, keepdims=True)
        m2 = jnp.mean(y2, axis=1, keepdims=True)
        mx = jnp.maximum(jnp.max(m1), jnp.max(m2))
        e1 = jnp.exp(m1 - mx)
        e2 = jnp.exp(m2 - mx)
        inv = 1.0 / (jnp.sum(e1) + jnp.sum(e2))
        o_ref[i] = (e1 * inv) * y1 + (e2 * inv) * y2


def kernel(x, wsq, wg, b_gwc, masks):
    N, C, H, W = x.shape
    S = H * W
    uq = wsq.shape[0] - C
    n_taps = masks.shape[0]
    kk = int(round(n_taps ** 0.5))
    kq = n_taps * uq

    nblk = 4 if N % 4 == 0 else 1

    xr = x.reshape(N, C, S)

    body = functools.partial(_cru_body, uq, H, W, kk, nblk)

    est = (2 * nblk * C * S * 4 + 2 * nblk * C * S * 4 + kq * S * 2
           + 4 * C * S * 4 + n_taps * S * 2 + (uq + C) * C * 2
           + C * kq * 2 + C * 4)
    vmem_limit = int(min(max(2 * est, 32 * 1024 * 1024),
                         int(64 * 1024 * 1024 * 0.9)))

    def _call(xl, wsql, wgl, bl, ml):
        Nl = xl.shape[0]
        G = Nl // nblk
        return pl.pallas_call(
            body,
            out_shape=jax.ShapeDtypeStruct((Nl, C, S), jnp.float32),
            grid=(G,),
            in_specs=[
                pl.BlockSpec((nblk, C, S), lambda b: (b, 0, 0)),
                pl.BlockSpec(wsq.shape, lambda b: (0, 0)),
                pl.BlockSpec(wg.shape, lambda b: (0, 0)),
                pl.BlockSpec(b_gwc.shape, lambda b: (0, 0)),
                pl.BlockSpec(masks.shape, lambda b: (0, 0)),
            ],
            out_specs=pl.BlockSpec((nblk, C, S), lambda b: (b, 0, 0)),
            compiler_params=pltpu.CompilerParams(
                dimension_semantics=("arbitrary",),
                vmem_limit_bytes=vmem_limit),
        )(xl, wsql, wgl, bl, ml)

    devs = jax.devices()
    ndev = 2 if (len(devs) >= 2 and N % (2 * nblk) == 0) else 1
    if ndev > 1:
        mesh = Mesh(np.array(devs[:ndev]), ("b",))
        fsh = jax.shard_map(_call, mesh=mesh,
                            in_specs=(P("b"), P(), P(), P(), P()),
                            out_specs=P("b"), check_vma=False)
        out = fsh(xr, wsq, wg, b_gwc, masks)
    else:
        out = _call(xr, wsq, wg, b_gwc, masks)

    return out.reshape(N, C, H, W)


# speedup vs baseline: 3.9815x; 3.9815x over previous
import functools

import jax
import jax.numpy as jnp
import numpy as np
from jax.experimental import pallas as pl
from jax.experimental.pallas import tpu as pltpu
from jax.sharding import Mesh, PartitionSpec as P


def _cru_body(uq, H, W, kk, nblk, x_ref, wsq_ref, wg_ref, bias_ref, mask_ref,
              o_ref):
    S = H * W
    pad = kk // 2
    wsq = wsq_ref[...]
    wg = wg_ref[...]
    bias = bias_ref[...]

    for i in range(nblk):
        x = x_ref[i]

        ul = jnp.dot(wsq, x, preferred_element_type=jnp.float32)
        u = ul[:uq, :].astype(jnp.bfloat16)
        y2 = ul[uq:, :]

        taps = []
        t = 0
        for ky in range(kk):
            for kx in range(kk):
                dy, dx = ky - pad, kx - pad
                if dy == 0 and dx == 0:
                    taps.append(u)
                else:
                    shift = (-(dy * W + dx)) % S
                    rolled = pltpu.roll(u, shift=shift, axis=1)
                    taps.append(rolled * mask_ref[t:t + 1, :])
                t += 1
        ucat = jnp.concatenate(taps, axis=0)

        y1 = jnp.dot(wg, ucat, preferred_element_type=jnp.float32) + bias

        m1 = jnp.mean(y1, axis=1, keepdims=True)
        m2 = jnp.mean(y2, axis=1, keepdims=True)
        mx = jnp.maximum(jnp.max(m1), jnp.max(m2))
        e1 = jnp.exp(m1 - mx)
        e2 = jnp.exp(m2 - mx)
        inv = 1.0 / (jnp.sum(e1) + jnp.sum(e2))
        o_ref[i] = (e1 * inv) * y1 + (e2 * inv) * y2


def kernel(x, wsq, wg, b_gwc, masks):
    N, C, H, W = x.shape
    S = H * W
    uq = wsq.shape[0] - C
    n_taps = masks.shape[0]
    kk = int(round(n_taps ** 0.5))
    kq = n_taps * uq

    nblk = 4 if N % 4 == 0 else 1

    xr = x.astype(jnp.bfloat16).reshape(N, C, S)

    body = functools.partial(_cru_body, uq, H, W, kk, nblk)

    est = (2 * nblk * C * S * 4 + 2 * nblk * C * S * 4 + kq * S * 2
           + 4 * C * S * 4 + n_taps * S * 2 + (uq + C) * C * 2
           + C * kq * 2 + C * 4)
    vmem_limit = int(min(max(2 * est, 32 * 1024 * 1024),
                         int(64 * 1024 * 1024 * 0.9)))

    def _call(xl, wsql, wgl, bl, ml):
        Nl = xl.shape[0]
        G = Nl // nblk
        return pl.pallas_call(
            body,
            out_shape=jax.ShapeDtypeStruct((Nl, C, S), jnp.float32),
            grid=(G,),
            in_specs=[
                pl.BlockSpec((nblk, C, S), lambda b: (b, 0, 0)),
                pl.BlockSpec(wsq.shape, lambda b: (0, 0)),
                pl.BlockSpec(wg.shape, lambda b: (0, 0)),
                pl.BlockSpec(b_gwc.shape, lambda b: (0, 0)),
                pl.BlockSpec(masks.shape, lambda b: (0, 0)),
            ],
            out_specs=pl.BlockSpec((nblk, C, S), lambda b: (b, 0, 0)),
            compiler_params=pltpu.CompilerParams(
                dimension_semantics=("arbitrary",),
                vmem_limit_bytes=vmem_limit),
        )(xl, wsql, wgl, bl, ml)

    out = _call(xr, wsq, wg, b_gwc, masks)

    return out.reshape(N, C, H, W)


# speedup vs baseline: 4.2564x; 1.0690x over previous
import functools

import jax
import jax.numpy as jnp
import numpy as np
from jax.experimental import pallas as pl
from jax.experimental.pallas import tpu as pltpu
from jax.sharding import Mesh, PartitionSpec as P


def _cru_body(uq, H, W, kk, nblk, xa_ref, xb_ref, wsq_ref, wg_ref, bias_ref,
              mask_ref, o_ref):
    S = H * W
    pad = kk // 2
    wsq = wsq_ref[...]
    wg = wg_ref[...]
    bias = bias_ref[...]
    Ch = xa_ref.shape[1]

    for i in range(nblk):
        xa = xa_ref[i].astype(jnp.bfloat16)
        xb = xb_ref[i].astype(jnp.bfloat16)

        ul = (jnp.dot(wsq[:, :Ch], xa, preferred_element_type=jnp.float32)
              + jnp.dot(wsq[:, Ch:], xb, preferred_element_type=jnp.float32))
        u = ul[:uq, :].astype(jnp.bfloat16)
        y2 = ul[uq:, :]

        taps = []
        t = 0
        for ky in range(kk):
            for kx in range(kk):
                dy, dx = ky - pad, kx - pad
                if dy == 0 and dx == 0:
                    taps.append(u)
                else:
                    shift = (-(dy * W + dx)) % S
                    rolled = pltpu.roll(u, shift=shift, axis=1)
                    taps.append(rolled * mask_ref[t:t + 1, :])
                t += 1
        ucat = jnp.concatenate(taps, axis=0)

        y1 = jnp.dot(wg, ucat, preferred_element_type=jnp.float32) + bias

        m1 = jnp.mean(y1, axis=1, keepdims=True)
        m2 = jnp.mean(y2, axis=1, keepdims=True)
        mx = jnp.maximum(jnp.max(m1), jnp.max(m2))
        e1 = jnp.exp(m1 - mx)
        e2 = jnp.exp(m2 - mx)
        inv = 1.0 / (jnp.sum(e1) + jnp.sum(e2))
        o_ref[i] = (e1 * inv) * y1 + (e2 * inv) * y2


def kernel(x, wsq, wg, b_gwc, masks):
    N, C, H, W = x.shape
    S = H * W
    uq = wsq.shape[0] - C
    n_taps = masks.shape[0]
    kk = int(round(n_taps ** 0.5))
    kq = n_taps * uq

    nblk = 4 if N % 4 == 0 else 1

    xr = x.reshape(N, C, S)

    body = functools.partial(_cru_body, uq, H, W, kk, nblk)

    est = (2 * nblk * C * S * 4 + 2 * nblk * C * S * 4 + kq * S * 2
           + 4 * C * S * 4 + n_taps * S * 2 + (uq + C) * C * 2
           + C * kq * 2 + C * 4)
    vmem_limit = int(min(max(2 * est, 32 * 1024 * 1024),
                         int(64 * 1024 * 1024 * 0.9)))

    G = N // nblk
    out = pl.pallas_call(
        body,
        out_shape=jax.ShapeDtypeStruct((N, C, S), jnp.float32),
        grid=(G,),
        in_specs=[
            pl.BlockSpec((nblk, C // 2, S), lambda b: (b, 0, 0)),
            pl.BlockSpec((nblk, C // 2, S), lambda b: (b, 1, 0)),
            pl.BlockSpec(wsq.shape, lambda b: (0, 0)),
            pl.BlockSpec(wg.shape, lambda b: (0, 0)),
            pl.BlockSpec(b_gwc.shape, lambda b: (0, 0)),
            pl.BlockSpec(masks.shape, lambda b: (0, 0)),
        ],
        out_specs=pl.BlockSpec((nblk, C, S), lambda b: (b, 0, 0)),
        compiler_params=pltpu.CompilerParams(
            dimension_semantics=("arbitrary",),
            vmem_limit_bytes=vmem_limit),
    )(xr, xr, wsq, wg, b_gwc, masks)

    return out.reshape(N, C, H, W)


# speedup vs baseline: 4.3190x; 1.0147x over previous
import functools

import jax
import jax.numpy as jnp
from jax.experimental import pallas as pl
from jax.experimental.pallas import tpu as pltpu


def _cru_body(uq, H, W, kk, nblk, up_c, groups,
              x_ref, w1_ref, w2x_ref, wgc_ref, wgg_ref, bias_ref, mask_ref,
              o_ref):
    S = H * W
    pad = kk // 2
    centre = (kk * kk) // 2
    ipg = uq // groups
    C = wgc_ref.shape[0]
    opg = C // groups
    w1 = w1_ref[...]
    w2x = w2x_ref[...]
    wgc = wgc_ref[...]
    wgg = wgg_ref[...]
    bias = bias_ref[...]

    for i in range(nblk):
        x = x_ref[i].astype(jnp.bfloat16)

        u = jnp.dot(w1, x[:up_c], preferred_element_type=jnp.float32)
        ub = u.astype(jnp.bfloat16)
        y2 = jnp.dot(w2x, x[up_c:], preferred_element_type=jnp.float32)

        taps = [[] for _ in range(groups)]
        t = 0
        for ky in range(kk):
            for kx in range(kk):
                dy, dx = ky - pad, kx - pad
                if dy == 0 and dx == 0:
                    t += 1
                    continue
                shift = (-(dy * W + dx)) % S
                rolled = pltpu.roll(ub, shift=shift, axis=1)
                tap = rolled * mask_ref[t:t + 1, :]
                for g in range(groups):
                    taps[g].append(tap[g * ipg:(g + 1) * ipg])
                t += 1

        y1 = jnp.dot(wgc, ub, preferred_element_type=jnp.float32) + bias
        parts = []
        for g in range(groups):
            cat_g = jnp.concatenate(taps[g], axis=0)
            parts.append(jnp.dot(wgg[g * opg:(g + 1) * opg],
                                 cat_g, preferred_element_type=jnp.float32))
        y1 = y1 + jnp.concatenate(parts, axis=0)

        m1 = jnp.mean(y1, axis=1, keepdims=True)
        m2 = jnp.mean(y2, axis=1, keepdims=True)
        mx = jnp.maximum(jnp.max(m1), jnp.max(m2))
        e1 = jnp.exp(m1 - mx)
        e2 = jnp.exp(m2 - mx)
        inv = 1.0 / (jnp.sum(e1) + jnp.sum(e2))
        o_ref[i] = (e1 * inv) * y1 + (e2 * inv) * y2


def kernel(x, wsq, wg, b_gwc, masks):
    N, C, H, W = x.shape
    S = H * W
    uq = wsq.shape[0] - C
    n_taps = masks.shape[0]
    kk = int(round(n_taps ** 0.5))
    centre = n_taps // 2
    up_c = C // 2
    groups = 2
    ipg = uq // groups
    opg = C // groups

    nblk = 4 if N % 4 == 0 else 1
    G = N // nblk

    xr = x.reshape(N, C, S)

    w1 = wsq[:uq, :up_c]
    w2x = wsq[uq:, up_c:]
    wgc = wg[:, centre * uq:(centre + 1) * uq]
    cols = []
    for g in range(groups):
        gcols = []
        for t in range(n_taps):
            if t == centre:
                continue
            gcols.append(wg[g * opg:(g + 1) * opg,
                            t * uq + g * ipg:t * uq + (g + 1) * ipg])
        cols.append(jnp.concatenate(gcols, axis=1))
    wgg = jnp.concatenate(cols, axis=0)

    body = functools.partial(_cru_body, uq, H, W, kk, nblk, up_c, groups)

    est = (2 * nblk * C * S * 4 + 2 * nblk * C * S * 4
           + (n_taps - 1) * uq * S * 2 + 4 * C * S * 4 + n_taps * S * 2
           + (uq + C) * C * 2 + C * n_taps * uq * 2 + C * 8)
    vmem_limit = int(min(max(2 * est, 32 * 1024 * 1024),
                         int(64 * 1024 * 1024 * 0.9)))

    out = pl.pallas_call(
        body,
        out_shape=jax.ShapeDtypeStruct((N, C, S), jnp.float32),
        grid=(G,),
        in_specs=[
            pl.BlockSpec((nblk, C, S), lambda b: (b, 0, 0)),
            pl.BlockSpec(w1.shape, lambda b: (0, 0)),
            pl.BlockSpec(w2x.shape, lambda b: (0, 0)),
            pl.BlockSpec(wgc.shape, lambda b: (0, 0)),
            pl.BlockSpec(wgg.shape, lambda b: (0, 0)),
            pl.BlockSpec(b_gwc.shape, lambda b: (0, 0)),
            pl.BlockSpec(masks.shape, lambda b: (0, 0)),
        ],
        out_specs=pl.BlockSpec((nblk, C, S), lambda b: (b, 0, 0)),
        compiler_params=pltpu.CompilerParams(
            dimension_semantics=("arbitrary",),
            vmem_limit_bytes=vmem_limit),
    )(xr, w1, w2x, wgc, wgg, b_gwc, masks)

    return out.reshape(N, C, H, W)


# speedup vs baseline: 4.4453x; 1.0292x over previous
import functools

import jax
import jax.numpy as jnp
from jax.experimental import pallas as pl
from jax.experimental.pallas import tpu as pltpu

_DEPTH = 4


def _cru_body(uq, H, W, kk, nblk, G,
              x_hbm, wsq_ref, wg_ref, bias_ref, mask_ref, o_ref,
              xbuf, sem):
    S = H * W
    pad = kk // 2
    wsq = wsq_ref[...]
    wg = wg_ref[...]
    bias = bias_ref[...]
    b = pl.program_id(0)

    @pl.when(b == 0)
    def _():
        for j in range(_DEPTH - 1):
            pltpu.make_async_copy(
                x_hbm.at[pl.ds(j * nblk, nblk)], xbuf.at[j], sem.at[j]
            ).start()

    slot = jax.lax.rem(b, _DEPTH)
    pltpu.make_async_copy(xbuf.at[slot], xbuf.at[slot], sem.at[slot]).wait()

    @pl.when(b + (_DEPTH - 1) < G)
    def _():
        nslot = jax.lax.rem(b + (_DEPTH - 1), _DEPTH)
        pltpu.make_async_copy(
            x_hbm.at[pl.ds((b + (_DEPTH - 1)) * nblk, nblk)],
            xbuf.at[nslot], sem.at[nslot]
        ).start()

    for i in range(nblk):
        x = xbuf[slot, i].astype(jnp.bfloat16)

        ul = jnp.dot(wsq, x, preferred_element_type=jnp.float32)
        u = ul[:uq, :].astype(jnp.bfloat16)
        y2 = ul[uq:, :]

        taps = []
        t = 0
        for ky in range(kk):
            for kx in range(kk):
                dy, dx = ky - pad, kx - pad
                if dy == 0 and dx == 0:
                    taps.append(u)
                else:
                    shift = (-(dy * W + dx)) % S
                    rolled = pltpu.roll(u, shift=shift, axis=1)
                    taps.append(rolled * mask_ref[t:t + 1, :])
                t += 1
        ucat = jnp.concatenate(taps, axis=0)

        y1 = jnp.dot(wg, ucat, preferred_element_type=jnp.float32) + bias

        m1 = jnp.mean(y1, axis=1, keepdims=True)
        m2 = jnp.mean(y2, axis=1, keepdims=True)
        mx = jnp.maximum(jnp.max(m1), jnp.max(m2))
        e1 = jnp.exp(m1 - mx)
        e2 = jnp.exp(m2 - mx)
        inv = 1.0 / (jnp.sum(e1) + jnp.sum(e2))
        o_ref[i] = (e1 * inv) * y1 + (e2 * inv) * y2


def kernel(x, wsq, wg, b_gwc, masks):
    N, C, H, W = x.shape
    S = H * W
    uq = wsq.shape[0] - C
    n_taps = masks.shape[0]
    kk = int(round(n_taps ** 0.5))
    kq = n_taps * uq

    nblk = 4 if N % 4 == 0 and N // 4 >= _DEPTH else 1
    G = N // nblk

    xr = x.reshape(N, C, S)

    body = functools.partial(_cru_body, uq, H, W, kk, nblk, G)

    est = (_DEPTH * nblk * C * S * 4 + 2 * nblk * C * S * 4 + kq * S * 2
           + 4 * C * S * 4 + n_taps * S * 2 + (uq + C) * C * 2
           + C * kq * 2 + C * 4)
    vmem_limit = int(min(max(2 * est, 32 * 1024 * 1024),
                         int(64 * 1024 * 1024 * 0.9)))

    out = pl.pallas_call(
        body,
        out_shape=jax.ShapeDtypeStruct((N, C, S), jnp.float32),
        grid=(G,),
        in_specs=[
            pl.BlockSpec(memory_space=pl.ANY),
            pl.BlockSpec(wsq.shape, lambda b: (0, 0)),
            pl.BlockSpec(wg.shape, lambda b: (0, 0)),
            pl.BlockSpec(b_gwc.shape, lambda b: (0, 0)),
            pl.BlockSpec(masks.shape, lambda b: (0, 0)),
        ],
        out_specs=pl.BlockSpec((nblk, C, S), lambda b: (b, 0, 0)),
        scratch_shapes=[pltpu.VMEM((_DEPTH, nblk, C, S), jnp.float32),
                        pltpu.SemaphoreType.DMA((_DEPTH,))],
        compiler_params=pltpu.CompilerParams(
            dimension_semantics=("arbitrary",),
            vmem_limit_bytes=vmem_limit),
    )(xr, wsq, wg, b_gwc, masks)

    return out.reshape(N, C, H, W)
